# baseline (speedup 1.0000x reference)
import jax
import jax.numpy as jnp
from jax.experimental import pallas as pl
from jax.experimental.pallas import tpu as pltpu


def _router_kernel(x_ref, wr_ref, br_ref, sched_ref, cstep_ref):
    T, D = x_ref.shape
    E = wr_ref.shape[1]
    f32 = jnp.float32

    logits = jnp.dot(x_ref[...], wr_ref[...], preferred_element_type=f32)
    logits = logits + br_ref[...]

    iota_e = jax.lax.broadcasted_iota(jnp.int32, (T, E), 1).astype(f32)
    big = f32(E)
    m1 = jnp.max(logits, axis=1, keepdims=True)
    a1 = jnp.min(jnp.where(logits == m1, iota_e, big), axis=1, keepdims=True)
    neg = f32(-jnp.inf)
    masked = jnp.where(iota_e == a1, neg, logits)
    m2 = jnp.max(masked, axis=1, keepdims=True)
    a2 = jnp.min(jnp.where(masked == m2, iota_e, big), axis=1, keepdims=True)

    r = jnp.exp(m2 - m1)
    w1 = 1.0 / (1.0 + r)
    w2 = 1.0 - w1

    sel1 = iota_e == a1
    sel2 = iota_e == a2
    comb = jnp.where(sel1, w1, 0.0) + jnp.where(sel2, w2, 0.0)

    active = jnp.max(jnp.where(sel1 | sel2, 1.0, 0.0), axis=0, keepdims=True)
    ii = jax.lax.broadcasted_iota(jnp.int32, (E, E), 0)
    jj = jax.lax.broadcasted_iota(jnp.int32, (E, E), 1)
    upper = jnp.where(ii <= jj, f32(1.0), f32(0.0))
    cum = jnp.dot(active, upper, preferred_element_type=f32)
    total = jnp.sum(active)

    rank = jax.lax.broadcasted_iota(jnp.int32, (E, 1), 0).astype(f32) + 1.0
    is_active = active > 0.5
    m_pad = jnp.where((cum == rank) & is_active, 1.0, 0.0)
    rank_cl = jnp.minimum(rank, total)
    m_clamp = jnp.where((cum == rank_cl) & is_active, 1.0, 0.0)

    sched = jnp.sum(m_clamp * jj.astype(f32), axis=1)
    sched_ref[...] = jnp.reshape(sched, (1, E))
    cstep_ref[...] = jax.lax.dot_general(
        m_pad, comb, (((1,), (1,)), ((), ())), preferred_element_type=f32)


def _expert_kernel(sched_ref, x_ref, w1_ref, b1_ref, w2_ref, b2_ref,
                   w3_ref, b3_ref, cstep_ref, out_ref):
    g = pl.program_id(0)
    e = sched_ref[g]
    T = x_ref.shape[0]
    f32 = jnp.float32

    h1 = jnp.dot(x_ref[...], w1_ref[0], preferred_element_type=f32)
    h1 = h1 + b1_ref[pl.ds(e, 1), :]
    h2 = jnp.dot(h1, w2_ref[0], preferred_element_type=f32)
    h2 = h2 + b2_ref[pl.ds(e, 1), :]
    a = h2 * (1.0 / (1.0 + jnp.exp(-h2)))
    y = jnp.dot(a, w3_ref[0], preferred_element_type=f32)
    y = y + b3_ref[pl.ds(e, 1), :]

    c = cstep_ref[pl.ds(g, 1), :]
    ii = jax.lax.broadcasted_iota(jnp.int32, (T, T), 0)
    jj = jax.lax.broadcasted_iota(jnp.int32, (T, T), 1)
    dc = jnp.where(ii == jj, c, f32(0.0))
    contrib = jnp.dot(dc, y, preferred_element_type=f32)

    @pl.when(g == 0)
    def _():
        out_ref[...] = jnp.zeros_like(out_ref)

    out_ref[...] += contrib


def kernel(x, Wr, br, W1, b1, W2, b2, W3, b3):
    B, S, D = x.shape
    E = Wr.shape[1]
    H = W1.shape[2]
    T = B * S
    x2 = x.reshape(T, D)

    sched_f, cstep = pl.pallas_call(
        _router_kernel,
        out_shape=[
            jax.ShapeDtypeStruct((1, E), jnp.float32),
            jax.ShapeDtypeStruct((E, T), jnp.float32),
        ],
    )(x2, Wr, br.reshape(1, E))
    sched = sched_f.reshape(E).astype(jnp.int32)

    out = pl.pallas_call(
        _expert_kernel,
        grid_spec=pltpu.PrefetchScalarGridSpec(
            num_scalar_prefetch=1,
            grid=(E,),
            in_specs=[
                pl.BlockSpec((T, D), lambda g, s: (0, 0)),
                pl.BlockSpec((1, D, H), lambda g, s: (s[g], 0, 0)),
                pl.BlockSpec((E, H), lambda g, s: (0, 0)),
                pl.BlockSpec((1, H, H), lambda g, s: (s[g], 0, 0)),
                pl.BlockSpec((E, H), lambda g, s: (0, 0)),
                pl.BlockSpec((1, H, D), lambda g, s: (s[g], 0, 0)),
                pl.BlockSpec((E, D), lambda g, s: (0, 0)),
                pl.BlockSpec((E, T), lambda g, s: (0, 0)),
            ],
            out_specs=pl.BlockSpec((T, D), lambda g, s: (0, 0)),
        ),
        out_shape=jax.ShapeDtypeStruct((T, D), jnp.float32),
        compiler_params=pltpu.CompilerParams(
            dimension_semantics=("arbitrary",),
        ),
    )(sched, x2, W1, b1, W2, b2, W3, b3, cstep)

    return out.reshape(B, S, D)


# speedup vs baseline: 1.1058x; 1.1058x over previous
import jax
import jax.numpy as jnp
from jax.experimental import pallas as pl
from jax.experimental.pallas import tpu as pltpu


def _router_kernel(x_ref, wr_ref, br_ref, sched_ref, nact_ref, cstep_ref):
    T, D = x_ref.shape
    E = wr_ref.shape[1]
    f32 = jnp.float32

    logits = jnp.dot(x_ref[...], wr_ref[...], preferred_element_type=f32)
    logits = logits + br_ref[...]

    iota_e = jax.lax.broadcasted_iota(jnp.int32, (T, E), 1).astype(f32)
    big = f32(E)
    m1 = jnp.max(logits, axis=1, keepdims=True)
    a1 = jnp.min(jnp.where(logits == m1, iota_e, big), axis=1, keepdims=True)
    neg = f32(-jnp.inf)
    masked = jnp.where(iota_e == a1, neg, logits)
    m2 = jnp.max(masked, axis=1, keepdims=True)
    a2 = jnp.min(jnp.where(masked == m2, iota_e, big), axis=1, keepdims=True)

    r = jnp.exp(m2 - m1)
    w1 = 1.0 / (1.0 + r)
    w2 = 1.0 - w1

    sel1 = iota_e == a1
    sel2 = iota_e == a2
    comb = jnp.where(sel1, w1, 0.0) + jnp.where(sel2, w2, 0.0)

    active = jnp.max(jnp.where(sel1 | sel2, 1.0, 0.0), axis=0, keepdims=True)
    ii = jax.lax.broadcasted_iota(jnp.int32, (E, E), 0)
    jj = jax.lax.broadcasted_iota(jnp.int32, (E, E), 1)
    upper = jnp.where(ii <= jj, f32(1.0), f32(0.0))
    cum = jnp.dot(active, upper, preferred_element_type=f32)
    total = jnp.sum(active)

    rank = jax.lax.broadcasted_iota(jnp.int32, (E, 1), 0).astype(f32) + 1.0
    is_active = active > 0.5
    m_pad = jnp.where((cum == rank) & is_active, 1.0, 0.0)
    rank_cl = jnp.minimum(rank, total)
    m_clamp = jnp.where((cum == rank_cl) & is_active, 1.0, 0.0)

    sched = jnp.sum(m_clamp * jj.astype(f32), axis=1)
    sched_ref[...] = jnp.reshape(sched, (1, E)).astype(jnp.int32)
    nact_ref[...] = jnp.reshape(total, (1, 1)).astype(jnp.int32)
    cstep_ref[...] = jax.lax.dot_general(
        m_pad, comb, (((1,), (1,)), ((), ())), preferred_element_type=f32)


def _expert_kernel(sched_ref, nact_ref, x_ref, w1_ref, b1_ref, w2_ref, b2_ref,
                   w3_ref, b3_ref, cstep_ref, out_ref):
    g = pl.program_id(0)
    T = x_ref.shape[0]
    f32 = jnp.float32

    @pl.when(g == 0)
    def _():
        out_ref[...] = jnp.zeros_like(out_ref)

    @pl.when(g < nact_ref[0])
    def _():
        e = sched_ref[g]
        h1 = jnp.dot(x_ref[...], w1_ref[0], preferred_element_type=f32)
        h1 = h1 + b1_ref[pl.ds(e, 1), :]
        h2 = jnp.dot(h1, w2_ref[0], preferred_element_type=f32)
        h2 = h2 + b2_ref[pl.ds(e, 1), :]
        a = h2 * (1.0 / (1.0 + jnp.exp(-h2)))
        y = jnp.dot(a, w3_ref[0], preferred_element_type=f32)
        y = y + b3_ref[pl.ds(e, 1), :]

        c = cstep_ref[pl.ds(g, 1), :]
        ii = jax.lax.broadcasted_iota(jnp.int32, (T, T), 0)
        jj = jax.lax.broadcasted_iota(jnp.int32, (T, T), 1)
        dc = jnp.where(ii == jj, c, f32(0.0))
        out_ref[...] += jnp.dot(dc, y, preferred_element_type=f32)


def kernel(x, Wr, br, W1, b1, W2, b2, W3, b3):
    B, S, D = x.shape
    E = Wr.shape[1]
    H = W1.shape[2]
    T = B * S
    x2 = x.reshape(T, D)

    sched2, nact2, cstep = pl.pallas_call(
        _router_kernel,
        out_shape=[
            jax.ShapeDtypeStruct((1, E), jnp.int32),
            jax.ShapeDtypeStruct((1, 1), jnp.int32),
            jax.ShapeDtypeStruct((E, T), jnp.float32),
        ],
    )(x2, Wr, br.reshape(1, E))
    sched = sched2.reshape(E)
    nact = nact2.reshape(1)

    out = pl.pallas_call(
        _expert_kernel,
        grid_spec=pltpu.PrefetchScalarGridSpec(
            num_scalar_prefetch=2,
            grid=(E,),
            in_specs=[
                pl.BlockSpec((T, D), lambda g, s, n: (0, 0)),
                pl.BlockSpec((1, D, H), lambda g, s, n: (s[g], 0, 0)),
                pl.BlockSpec((E, H), lambda g, s, n: (0, 0)),
                pl.BlockSpec((1, H, H), lambda g, s, n: (s[g], 0, 0)),
                pl.BlockSpec((E, H), lambda g, s, n: (0, 0)),
                pl.BlockSpec((1, H, D), lambda g, s, n: (s[g], 0, 0)),
                pl.BlockSpec((E, D), lambda g, s, n: (0, 0)),
                pl.BlockSpec((E, T), lambda g, s, n: (0, 0)),
            ],
            out_specs=pl.BlockSpec((T, D), lambda g, s, n: (0, 0)),
        ),
        out_shape=jax.ShapeDtypeStruct((T, D), jnp.float32),
        compiler_params=pltpu.CompilerParams(
            dimension_semantics=("arbitrary",),
        ),
    )(sched, nact, x2, W1, b1, W2, b2, W3, b3, cstep)

    return out.reshape(B, S, D)
